# Initial kernel scaffold; baseline (speedup 1.0000x reference)
import jax
import jax.numpy as jnp
from jax.experimental import pallas as pl


def kernel(src, tgt, table):
    raise NotImplementedError("write your pallas kernel here")


# speedup vs baseline: 5.0279x; 5.0279x over previous
import functools
import math

import jax
import jax.numpy as jnp
import numpy as np
from jax import lax
from jax.experimental import pallas as pl
from jax.experimental.pallas import tpu as pltpu
from jax.experimental.pallas import tpu_sc as plsc

_B, _T, _F, _E = 1024, 50, 26, 16
_D = _F * _E
_N = _B * _T * _F
_SCALE = math.sqrt(float(_D))

_NW = 32
_PERIOD = _T * _F
_CHUNK_B = 4
_CHUNK = _CHUNK_B * _PERIOD
_SUB = 100
_NSUB = _CHUNK // _SUB
_NCHUNK = _N // (_NW * _CHUNK)


def _pe_rows():
    pe = np.zeros((_T, _D), dtype=np.float32)
    pos = np.arange(_T, dtype=np.float32)[:, None]
    denom = np.exp(np.arange(0, _D, 2, dtype=np.float32) * (-np.log(10000.0) / _D))
    pe[:, 0::2] = np.sin(pos * denom)
    pe[:, 1::2] = np.cos(pos * denom)
    return jnp.asarray(pe.reshape(_PERIOD, _E))


def _sc_body(src_hbm, tgt_hbm, pe_hbm, table_hbm, out_src, out_tgt,
             pe_v, idx_v, rows_v, sem):
    wid = lax.axis_index("s") * 2 + lax.axis_index("c")
    pltpu.sync_copy(pe_hbm, pe_v)

    for idx_hbm, out_hbm in ((src_hbm, out_src), (tgt_hbm, out_tgt)):
        def chunk_body(c, carry, idx_hbm=idx_hbm, out_hbm=out_hbm):
            cid = wid * _NCHUNK + c
            n0 = cid * _CHUNK
            pltpu.sync_copy(idx_hbm.at[cid], idx_v)

            def fire(j, carry2):
                pltpu.async_copy(table_hbm.at[idx_v.at[j]],
                                 rows_v.at[pl.ds(j * _SUB, _SUB)], sem)
                return carry2
            lax.fori_loop(0, _NSUB, fire, 0)
            pltpu.make_async_copy(out_hbm.at[pl.ds(0, _CHUNK)], rows_v,
                                  sem).wait()

            def comp(i, carry2):
                p = pe_v[i]
                for h in range(_CHUNK_B):
                    r = rows_v[h * _PERIOD + i]
                    rows_v[h * _PERIOD + i] = r * _SCALE + p
                return carry2
            lax.fori_loop(0, _PERIOD, comp, 0)

            pltpu.sync_copy(rows_v, out_hbm.at[pl.ds(n0, _CHUNK)])
            return carry
        lax.fori_loop(0, _NCHUNK, chunk_body, 0)


def kernel(src, tgt, table):
    src_i = src.reshape(_N // _CHUNK, _NSUB, _SUB)
    tgt_i = tgt.reshape(_N // _CHUNK, _NSUB, _SUB)
    pe = _pe_rows()

    mesh = plsc.VectorSubcoreMesh(core_axis_name="c", subcore_axis_name="s")
    f = functools.partial(
        pl.kernel,
        mesh=mesh,
        compiler_params=pltpu.CompilerParams(use_tc_tiling_on_sc=False),
        out_type=[jax.ShapeDtypeStruct((_N, _E), jnp.float32),
                  jax.ShapeDtypeStruct((_N, _E), jnp.float32)],
        scratch_types=[
            pltpu.VMEM((_PERIOD, _E), jnp.float32),
            pltpu.VMEM((_NSUB, _SUB), jnp.int32),
            pltpu.VMEM((_CHUNK, _E), jnp.float32),
            pltpu.SemaphoreType.DMA,
        ],
    )(_sc_body)
    out_s, out_t = f(src_i, tgt_i, pe, table)
    return out_s.reshape(_B, _T, _D), out_t.reshape(_B, _T, _D)
